# Initial kernel scaffold; baseline (speedup 1.0000x reference)
import jax
import jax.numpy as jnp
from jax.experimental import pallas as pl


def kernel(x, edge_index, W1, b1, W2, b2, Wlin, blin):
    raise NotImplementedError("write your pallas kernel here")


# speedup vs baseline: 18.3165x; 18.3165x over previous
import functools

import jax
import jax.numpy as jnp
from jax import lax
from jax.experimental import pallas as pl
from jax.experimental.pallas import tpu as pltpu
from jax.experimental.pallas import tpu_sc as plsc

NC = 2
NS = 16
NW = NC * NS
NNODE = 10000
NE = 320000
DIN = 128
DH = 80
NCLS = 7
EPT = NE // NW
CH = 40
NCHUNK = EPT // CH
RPS = 624
RPS_LAST = NNODE - RPS * (NS - 1)
DEGW = 16

_DOT_KW = dict(preferred_element_type=jnp.float32,
               precision=lax.Precision.HIGHEST)


def _rows_copy(sid, fn):
    @pl.when(sid < NS - 1)
    def _():
        fn(sid * RPS, RPS)

    @pl.when(sid == NS - 1)
    def _():
        fn((NS - 1) * RPS, RPS_LAST)


def _mesh():
    return plsc.VectorSubcoreMesh(core_axis_name="c", subcore_axis_name="s",
                                  num_cores=NC, num_subcores=NS)


_SC_PARAMS = pltpu.CompilerParams(use_tc_tiling_on_sc=False)


def _sc_deg(dst3, ones_deg, zeros_deg):
    @functools.partial(
        pl.kernel,
        out_type=jax.ShapeDtypeStruct((NC, NNODE, DEGW), jnp.float32),
        mesh=_mesh(),
        scratch_types=[
            pltpu.VMEM((NCHUNK, CH), jnp.int32),
            pltpu.VMEM((CH, DEGW), jnp.float32),
            pltpu.VMEM_SHARED((NNODE, DEGW), jnp.float32),
        ],
        compiler_params=_SC_PARAMS,
    )
    def k(dst_hbm, ones_hbm, z_hbm, out_hbm, div, ones_v, acc):
        cid = lax.axis_index("c")
        sid = lax.axis_index("s")
        wid = sid * NC + cid
        pltpu.sync_copy(dst_hbm.at[wid], div)
        pltpu.sync_copy(ones_hbm, ones_v)
        _rows_copy(sid, lambda s, n: pltpu.sync_copy(
            z_hbm.at[pl.ds(s, n)], acc.at[pl.ds(s, n)]))
        plsc.subcore_barrier()

        @pl.loop(0, NCHUNK)
        def _(j):
            pltpu.sync_copy(ones_v, acc.at[div.at[j]], add=True)

        plsc.subcore_barrier()
        _rows_copy(sid, lambda s, n: pltpu.sync_copy(
            acc.at[pl.ds(s, n)], out_hbm.at[cid, pl.ds(s, n)]))

    return k(dst3, ones_deg, zeros_deg)


def _sc_scatter(hs, src3, dst3, zeros_f):
    @functools.partial(
        pl.kernel,
        out_type=jax.ShapeDtypeStruct((NC, NNODE, DH), jnp.float32),
        mesh=_mesh(),
        scratch_types=[
            pltpu.VMEM((NCHUNK, CH), jnp.int32),
            pltpu.VMEM((NCHUNK, CH), jnp.int32),
            pltpu.VMEM((CH, DH), jnp.float32),
            pltpu.VMEM((CH, DH), jnp.float32),
            pltpu.VMEM_SHARED((NNODE, DH), jnp.float32),
            pltpu.SemaphoreType.DMA,
            pltpu.SemaphoreType.DMA,
        ],
        compiler_params=_SC_PARAMS,
    )
    def k(hs_hbm, src_hbm, dst_hbm, z_hbm, out_hbm,
          siv, div, b0, b1, acc, sg0, sg1):
        cid = lax.axis_index("c")
        sid = lax.axis_index("s")
        wid = sid * NC + cid
        pltpu.sync_copy(src_hbm.at[wid], siv)
        pltpu.sync_copy(dst_hbm.at[wid], div)
        _rows_copy(sid, lambda s, n: pltpu.sync_copy(
            z_hbm.at[pl.ds(s, n)], acc.at[pl.ds(s, n)]))
        plsc.subcore_barrier()

        pltpu.async_copy(hs_hbm.at[siv.at[0]], b0, sg0)

        @pl.loop(0, NCHUNK, step=2)
        def _(j):
            pltpu.make_async_copy(hs_hbm.at[siv.at[j]], b0, sg0).wait()
            pltpu.async_copy(hs_hbm.at[siv.at[j + 1]], b1, sg1)
            pltpu.sync_copy(b0, acc.at[div.at[j]], add=True)
            pltpu.make_async_copy(hs_hbm.at[siv.at[j + 1]], b1, sg1).wait()

            @pl.when(j + 2 < NCHUNK)
            def _():
                pltpu.async_copy(hs_hbm.at[siv.at[j + 2]], b0, sg0)

            pltpu.sync_copy(b1, acc.at[div.at[j + 1]], add=True)

        plsc.subcore_barrier()
        _rows_copy(sid, lambda s, n: pltpu.sync_copy(
            acc.at[pl.ds(s, n)], out_hbm.at[cid, pl.ds(s, n)]))

    return k(hs, src3, dst3, zeros_f)


_BM = 1000


def _tc_matmul(x, w):
    m, kdim = x.shape
    n = w.shape[1]

    def body(x_ref, w_ref, o_ref):
        o_ref[...] = lax.dot_general(x_ref[...], w_ref[...],
                                     (((1,), (0,)), ((), ())), **_DOT_KW)

    return pl.pallas_call(
        body, grid=(m // _BM,),
        in_specs=[pl.BlockSpec((_BM, kdim), lambda i: (i, 0)),
                  pl.BlockSpec((kdim, n), lambda i: (0, 0))],
        out_specs=pl.BlockSpec((_BM, n), lambda i: (i, 0)),
        out_shape=jax.ShapeDtypeStruct((m, n), jnp.float32))(x, w)


def _tc_scale(deg_parts, h):
    def body(dp_ref, h_ref, hs_ref, dinv_ref):
        deg = dp_ref[0, :, 0:1] + dp_ref[1, :, 0:1] + 1.0
        dinv = lax.rsqrt(deg)
        hs_ref[...] = h_ref[...] * dinv
        dinv_ref[...] = dinv

    return pl.pallas_call(
        body, grid=(NNODE // _BM,),
        in_specs=[pl.BlockSpec((NC, _BM, DEGW), lambda i: (0, i, 0)),
                  pl.BlockSpec((_BM, DH), lambda i: (i, 0))],
        out_specs=[pl.BlockSpec((_BM, DH), lambda i: (i, 0)),
                   pl.BlockSpec((_BM, 1), lambda i: (i, 0))],
        out_shape=[jax.ShapeDtypeStruct((NNODE, DH), jnp.float32),
                   jax.ShapeDtypeStruct((NNODE, 1), jnp.float32)],
    )(deg_parts, h)


def _tc_layer(acc, hs, dinv, b, w):
    n = w.shape[1]

    def body(a_ref, hs_ref, dinv_ref, b_ref, w_ref, o_ref):
        s = (a_ref[0] + a_ref[1] + hs_ref[...]) * dinv_ref[...] + b_ref[...]
        z = jnp.maximum(s, 0.0)
        o_ref[...] = lax.dot_general(z, w_ref[...],
                                     (((1,), (0,)), ((), ())),
                                     **_DOT_KW) * dinv_ref[...]

    return pl.pallas_call(
        body, grid=(NNODE // _BM,),
        in_specs=[pl.BlockSpec((NC, _BM, DH), lambda i: (0, i, 0)),
                  pl.BlockSpec((_BM, DH), lambda i: (i, 0)),
                  pl.BlockSpec((_BM, 1), lambda i: (i, 0)),
                  pl.BlockSpec((1, DH), lambda i: (0, 0)),
                  pl.BlockSpec((DH, n), lambda i: (0, 0))],
        out_specs=pl.BlockSpec((_BM, n), lambda i: (i, 0)),
        out_shape=jax.ShapeDtypeStruct((NNODE, n), jnp.float32),
    )(acc, hs, dinv, b, w)


def _tc_head(acc, hs, dinv, b, wlin, blin):
    def body(a_ref, hs_ref, dinv_ref, b_ref, w_ref, bl_ref, o_ref):
        s = (a_ref[0] + a_ref[1] + hs_ref[...]) * dinv_ref[...] + b_ref[...]
        z = jnp.maximum(s, 0.0)
        logits = lax.dot_general(z, w_ref[...], (((1,), (0,)), ((), ())),
                                 **_DOT_KW) + bl_ref[...]
        m = jnp.max(logits, axis=1, keepdims=True)
        shifted = logits - m
        lse = jnp.log(jnp.sum(jnp.exp(shifted), axis=1, keepdims=True))
        o_ref[...] = shifted - lse

    return pl.pallas_call(
        body, grid=(NNODE // _BM,),
        in_specs=[pl.BlockSpec((NC, _BM, DH), lambda i: (0, i, 0)),
                  pl.BlockSpec((_BM, DH), lambda i: (i, 0)),
                  pl.BlockSpec((_BM, 1), lambda i: (i, 0)),
                  pl.BlockSpec((1, DH), lambda i: (0, 0)),
                  pl.BlockSpec((DH, NCLS), lambda i: (0, 0)),
                  pl.BlockSpec((1, NCLS), lambda i: (0, 0))],
        out_specs=pl.BlockSpec((_BM, NCLS), lambda i: (i, 0)),
        out_shape=jax.ShapeDtypeStruct((NNODE, NCLS), jnp.float32),
    )(acc, hs, dinv, b, wlin, blin)


def kernel(x, edge_index, W1, b1, W2, b2, Wlin, blin):
    src3 = edge_index[0].astype(jnp.int32).reshape(NW, NCHUNK, CH)
    dst3 = edge_index[1].astype(jnp.int32).reshape(NW, NCHUNK, CH)
    ones_deg = jnp.ones((CH, DEGW), jnp.float32)
    zeros_deg = jnp.zeros((NNODE, DEGW), jnp.float32)
    zeros_f = jnp.zeros((NNODE, DH), jnp.float32)

    deg_parts = _sc_deg(dst3, ones_deg, zeros_deg)
    h1 = _tc_matmul(x, W1)
    hs1, dinv = _tc_scale(deg_parts, h1)
    acc1 = _sc_scatter(hs1, src3, dst3, zeros_f)
    hs2 = _tc_layer(acc1, hs1, dinv, b1.reshape(1, DH), W2)
    acc2 = _sc_scatter(hs2, src3, dst3, zeros_f)
    return _tc_head(acc2, hs2, dinv, b2.reshape(1, DH),
                    Wlin, blin.reshape(1, NCLS))
